# baseline (speedup 1.0000x reference)
import jax
import jax.numpy as jnp
from jax.experimental import pallas as pl
import jax.experimental.pallas.tpu as pltpu

_C = 96


def _gather_kernel(idx_ref, x_ref, o_ref, sel_ref, sem):
    def init(j, carry):
        sel_ref[j] = 0
        return carry

    jax.lax.fori_loop(0, _C, init, 0)

    def body(i, count):
        nz = idx_ref[i] != 0.0

        @pl.when(nz)
        def _():
            sel_ref[count] = i

        return count + nz.astype(jnp.int32)

    jax.lax.fori_loop(0, _C, body, 0)

    for j in range(_C):
        d = sel_ref[j]
        pltpu.make_async_copy(
            x_ref.at[:, pl.ds(d, 1)], o_ref.at[:, pl.ds(j, 1)], sem
        ).start()
    for j in range(_C):
        pltpu.make_async_copy(
            x_ref.at[:, pl.ds(0, 1)], o_ref.at[:, pl.ds(j, 1)], sem
        ).wait()


@jax.jit
def kernel(input_tensor, indexes):
    b, c, h, w = input_tensor.shape
    return pl.pallas_call(
        _gather_kernel,
        in_specs=[
            pl.BlockSpec(memory_space=pltpu.SMEM),
            pl.BlockSpec(memory_space=pltpu.HBM),
        ],
        out_specs=pl.BlockSpec(memory_space=pltpu.HBM),
        out_shape=jax.ShapeDtypeStruct((b, c, h, w), jnp.float32),
        scratch_shapes=[pltpu.SMEM((c,), jnp.int32), pltpu.SemaphoreType.DMA],
    )(indexes, input_tensor)


# speedup vs baseline: 10.8628x; 10.8628x over previous
import jax
import jax.numpy as jnp
from jax.experimental import pallas as pl
import jax.experimental.pallas.tpu as pltpu

_C = 96
_NSLOT = 8
_LOOK = 4


def _gather_kernel(idx_ref, x_ref, o_ref, sel_ref, buf_ref, in_sems, out_sems):
    def init(j, carry):
        sel_ref[j] = 0
        return carry

    jax.lax.fori_loop(0, _C, init, 0)

    def body(i, count):
        nz = idx_ref[i] != 0.0

        @pl.when(nz)
        def _():
            sel_ref[count] = i

        return count + nz.astype(jnp.int32)

    jax.lax.fori_loop(0, _C, body, 0)

    def copy_in(j, s):
        return pltpu.make_async_copy(
            x_ref.at[:, pl.ds(sel_ref[j], 1)], buf_ref.at[s], in_sems.at[s]
        )

    def copy_out(j, s):
        return pltpu.make_async_copy(
            buf_ref.at[s], o_ref.at[:, pl.ds(j, 1)], out_sems.at[s]
        )

    for t in range(_C + _LOOK):
        jin = t
        jout = t - _LOOK
        if jin < _C:
            s = jin % _NSLOT
            if jin >= _NSLOT:
                copy_out(jin - _NSLOT, s).wait()
            copy_in(jin, s).start()
        if 0 <= jout:
            s = jout % _NSLOT
            copy_in(jout, s).wait()
            copy_out(jout, s).start()
    for j in range(_C - _NSLOT, _C):
        copy_out(j, j % _NSLOT).wait()


@jax.jit
def kernel(input_tensor, indexes):
    b, c, h, w = input_tensor.shape
    hw = h * w
    lanes = 128
    sub = hw // lanes
    x = input_tensor.reshape(b, c, sub, lanes)
    out = pl.pallas_call(
        _gather_kernel,
        in_specs=[
            pl.BlockSpec(memory_space=pltpu.SMEM),
            pl.BlockSpec(memory_space=pltpu.HBM),
        ],
        out_specs=pl.BlockSpec(memory_space=pltpu.HBM),
        out_shape=jax.ShapeDtypeStruct((b, c, sub, lanes), jnp.float32),
        scratch_shapes=[
            pltpu.SMEM((c,), jnp.int32),
            pltpu.VMEM((_NSLOT, b, 1, sub, lanes), jnp.float32),
            pltpu.SemaphoreType.DMA((_NSLOT,)),
            pltpu.SemaphoreType.DMA((_NSLOT,)),
        ],
    )(indexes, x)
    return out.reshape(b, c, h, w)
